# baseline (speedup 1.0000x reference)
import functools

import jax
import jax.numpy as jnp
from jax import lax
from jax.experimental import pallas as pl
from jax.experimental.pallas import tpu as pltpu
from jax.experimental.pallas import tpu_sc as plsc

VOCAB_ = 1000000
EMBED = 64
B_ = 16384
C_ = 20

_NC = 2
_NS = 16
_NW = _NC * _NS
_BPW = B_ // _NW
_CB = 32
_NCHUNK = _BPW // _CB
_G = 4


def _sc_kernel(tgt_tab, ctx_tab, tidx_hbm, cidx_hbm, out_hbm,
               tidx_v, trows_v, cidx_v, crows_v, out_v, accmat_v, sem):
    wid = lax.axis_index("s") * _NC + lax.axis_index("c")
    base = wid * _BPW
    lane = lax.broadcasted_iota(jnp.int32, (16,), 0)
    sidx = lane * (_G * C_)

    pltpu.sync_copy(tidx_hbm.at[pl.ds(base, _BPW)], tidx_v)
    pltpu.async_copy(tgt_tab.at[tidx_v], trows_v, sem).wait()

    def chunk_body(i, _):
        flat = (base + i * _CB) * C_
        pltpu.sync_copy(cidx_hbm.at[pl.ds(flat, _CB * C_)], cidx_v)
        pltpu.async_copy(ctx_tab.at[cidx_v], crows_v, sem).wait()

        def group_body(g, _):
            for j in range(_G):
                b = g * _G + j
                gb = i * _CB + b
                t = [trows_v[gb, pl.ds(16 * m, 16)] for m in range(4)]
                for c in range(C_):
                    r = b * C_ + c
                    acc = crows_v[r, pl.ds(0, 16)] * t[0]
                    acc += crows_v[r, pl.ds(16, 16)] * t[1]
                    acc += crows_v[r, pl.ds(32, 16)] * t[2]
                    acc += crows_v[r, pl.ds(48, 16)] * t[3]
                    plsc.store_scatter(accmat_v, [sidx + (j * C_ + c)], acc)
            for k in range(_G * C_ // 16):
                s = accmat_v[pl.ds(16 * k, 16)]
                for m in range(1, 16):
                    s += accmat_v[pl.ds(m * _G * C_ + 16 * k, 16)]
                out_v[pl.ds(g * _G * C_ + 16 * k, 16)] = s
            return _

        lax.fori_loop(0, _CB // _G, group_body, None)
        pltpu.sync_copy(out_v, out_hbm.at[pl.ds(flat, _CB * C_)])
        return _

    lax.fori_loop(0, _NCHUNK, chunk_body, None)


@jax.jit
def _run(target_words, context_flat, target_table, context_table):
    mesh = plsc.VectorSubcoreMesh(core_axis_name="c", subcore_axis_name="s")
    k = functools.partial(
        pl.kernel,
        mesh=mesh,
        compiler_params=pltpu.CompilerParams(
            needs_layout_passes=False, use_tc_tiling_on_sc=False
        ),
        out_type=jax.ShapeDtypeStruct((B_ * C_,), jnp.float32),
        scratch_types=[
            pltpu.VMEM((_BPW,), jnp.int32),
            pltpu.VMEM((_BPW, EMBED), jnp.float32),
            pltpu.VMEM((_CB * C_,), jnp.int32),
            pltpu.VMEM((_CB * C_, EMBED), jnp.float32),
            pltpu.VMEM((_CB * C_,), jnp.float32),
            pltpu.VMEM((16 * _G * C_,), jnp.float32),
            pltpu.SemaphoreType.DMA,
        ],
    )(_sc_kernel)
    return k(target_table, context_table, target_words, context_flat)


def kernel(target_words, context_words, target_table, context_table):
    context_flat = context_words.reshape(-1)
    return _run(target_words, context_flat, target_table, context_table).reshape(B_, C_)
